# Initial kernel scaffold; baseline (speedup 1.0000x reference)
import jax
import jax.numpy as jnp
from jax.experimental import pallas as pl


def kernel(x, edge_index, edge_type, W1, root1, b1, W2, root2, b2):
    raise NotImplementedError("write your pallas kernel here")


# speedup vs baseline: 17.2851x; 17.2851x over previous
import functools

import jax
import jax.numpy as jnp
from jax import lax
from jax.experimental import pallas as pl
from jax.experimental.pallas import tpu as pltpu
from jax.experimental.pallas import tpu_sc as plsc

NC = 2
NS = 16
L = 16

GRP = 80
MAC = 2000
NCH = MAC // GRP


def _zero_f32(ref, n):
    def body(j, _):
        ref[pl.ds(16 * j, 16)] = jnp.zeros((16,), jnp.float32)
        return 0
    lax.fori_loop(0, n // 16, body, 0)


def _make_sc_agg(E, N, R, compute_w):
    EPT = E // (NC * NS)
    assert EPT % MAC == 0
    NMAC = EPT // MAC
    EPC = E // NS
    assert EPC % MAC == 0
    NMAC_CNT = EPC // MAC
    CNT_PAD = ((R * N + NS * 16 - 1) // (NS * 16)) * (NS * 16)
    ZROWS = CNT_PAD // NS
    ACK = (N // NS) & ~7
    REM = N - ACK * NS

    mesh = plsc.VectorSubcoreMesh(core_axis_name="c", subcore_axis_name="s",
                                  num_cores=NC, num_subcores=NS)

    out_type = [jax.ShapeDtypeStruct((NC * N, 128), jnp.float32)]
    if compute_w:
        out_type.append(jax.ShapeDtypeStruct((E,), jnp.float32))

    scratch = dict(
        acc_sh=pltpu.VMEM_SHARED((N, 128), jnp.float32),
        rows_v=pltpu.VMEM((GRP, 128), jnp.float32),
        etv=pltpu.VMEM((MAC,), jnp.int32),
        srcv=pltpu.VMEM((MAC,), jnp.int32),
        dstv=pltpu.VMEM((NCH, GRP), jnp.int32),
        gidx=pltpu.VMEM((NCH, GRP), jnp.int32),
        wv=pltpu.VMEM((MAC,), jnp.float32),
        zb=pltpu.VMEM((ZROWS,), jnp.float32),
    )
    if compute_w:
        scratch.update(
            cnt_sh=pltpu.VMEM_SHARED((CNT_PAD,), jnp.float32),
            ones_v=pltpu.VMEM((GRP,), jnp.float32),
            cw_v=pltpu.VMEM((MAC,), jnp.float32),
        )

    def body(y_hbm, src_hbm, dst_hbm, et_hbm, w_hbm, accp_hbm, w_out_hbm,
             acc_sh, rows_v, etv, srcv, dstv, gidx, wv, zb,
             cnt_sh=None, ones_v=None, cw_v=None):
        c = lax.axis_index("c")
        s = lax.axis_index("s")
        tile = c * NS + s
        ebase = tile * EPT

        def zrow(j, _):
            for q in range(8):
                rows_v[j, pl.ds(16 * q, 16)] = jnp.zeros((16,), jnp.float32)
            return 0
        lax.fori_loop(0, GRP, zrow, 0)
        a0 = s * ACK
        done = 0
        while done < ACK:
            n = min(GRP, ACK - done)
            pltpu.sync_copy(rows_v.at[pl.ds(0, n)], acc_sh.at[pl.ds(a0 + done, n)])
            done += n
        if REM:
            @pl.when(s == 0)
            def _():
                pltpu.sync_copy(rows_v.at[pl.ds(0, REM)],
                                acc_sh.at[pl.ds(NS * ACK, REM)])
        if compute_w:
            _zero_f32(zb, ZROWS)
            pltpu.sync_copy(zb, cnt_sh.at[pl.ds(s * ZROWS, ZROWS)])
            def ofill(j, _):
                ones_v[pl.ds(16 * j, 16)] = jnp.ones((16,), jnp.float32)
                return 0
            lax.fori_loop(0, GRP // 16, ofill, 0)
        plsc.subcore_barrier()

        def load_cidx(base):
            pltpu.sync_copy(et_hbm.at[pl.ds(base, MAC)], etv)
            def dch(ch, _):
                pltpu.sync_copy(dst_hbm.at[pl.ds(base + GRP * ch, GRP)],
                                dstv.at[ch])
                def cg(jj, _):
                    col = 16 * jj
                    e16 = etv[pl.ds(GRP * ch + col, 16)]
                    d16 = dstv[ch, pl.ds(col, 16)]
                    gidx[ch, pl.ds(col, 16)] = e16 * N + d16
                    return 0
                lax.fori_loop(0, GRP // 16, cg, 0)
                return 0
            lax.fori_loop(0, NCH, dch, 0)

        if compute_w:
            def count_mac(m, _):
                base = s * EPC + m * MAC
                load_cidx(base)
                def cch(ch, _):
                    pltpu.sync_copy(ones_v, cnt_sh.at[gidx.at[ch]], add=True)
                    return 0
                lax.fori_loop(0, NCH, cch, 0)
                return 0
            lax.fori_loop(0, NMAC_CNT, count_mac, 0)
            plsc.subcore_barrier()

            def w_mac(m, _):
                base = ebase + m * MAC
                load_cidx(base)
                def wch(ch, _):
                    pltpu.sync_copy(cnt_sh.at[gidx.at[ch]],
                                    cw_v.at[pl.ds(GRP * ch, GRP)])
                    return 0
                lax.fori_loop(0, NCH, wch, 0)
                def wcomp(j, _):
                    cvec = cw_v[pl.ds(16 * j, 16)]
                    wv[pl.ds(16 * j, 16)] = 1.0 / jnp.maximum(cvec, 1.0)
                    return 0
                lax.fori_loop(0, MAC // 16, wcomp, 0)
                pltpu.sync_copy(wv, w_out_hbm.at[pl.ds(base, MAC)])
                return 0
            lax.fori_loop(0, NMAC, w_mac, 0)
            w_src = w_out_hbm
        else:
            w_src = w_hbm

        def main_mac(m, _):
            base = ebase + m * MAC
            pltpu.sync_copy(et_hbm.at[pl.ds(base, MAC)], etv)
            pltpu.sync_copy(src_hbm.at[pl.ds(base, MAC)], srcv)
            pltpu.sync_copy(w_src.at[pl.ds(base, MAC)], wv)
            def mch(ch, _):
                pltpu.sync_copy(dst_hbm.at[pl.ds(base + GRP * ch, GRP)],
                                dstv.at[ch])
                def ig(jj, _):
                    col = 16 * jj
                    e16 = etv[pl.ds(GRP * ch + col, 16)]
                    s16 = srcv[pl.ds(GRP * ch + col, 16)]
                    gidx[ch, pl.ds(col, 16)] = e16 * N + s16
                    return 0
                lax.fori_loop(0, GRP // 16, ig, 0)
                pltpu.sync_copy(y_hbm.at[gidx.at[ch]], rows_v)
                def scale(jj, _):
                    w16 = wv[pl.ds(GRP * ch + 16 * jj, 16)]
                    for t in range(16):
                        wb = w16.at[jnp.full((16,), t, jnp.int32)].get(
                            mode="promise_in_bounds")
                        j = 16 * jj + t
                        for q in range(8):
                            rows_v[j, pl.ds(16 * q, 16)] = (
                                rows_v[j, pl.ds(16 * q, 16)] * wb)
                    return 0
                lax.fori_loop(0, GRP // 16, scale, 0)
                pltpu.sync_copy(rows_v, acc_sh.at[dstv.at[ch]], add=True)
                return 0
            lax.fori_loop(0, NCH, mch, 0)
            return 0
        lax.fori_loop(0, NMAC, main_mac, 0)
        plsc.subcore_barrier()

        done = 0
        while done < ACK:
            n = min(GRP, ACK - done)
            pltpu.sync_copy(acc_sh.at[pl.ds(a0 + done, n)],
                            accp_hbm.at[pl.ds(c * N + a0 + done, n)])
            done += n
        if REM:
            @pl.when(s == 0)
            def _():
                pltpu.sync_copy(acc_sh.at[pl.ds(NS * ACK, REM)],
                                accp_hbm.at[pl.ds(c * N + NS * ACK, REM)])

    if compute_w:
        def body_w(y, src, dst, et, accp, w_out, *scr):
            return body(y, src, dst, et, None, accp, w_out, *scr)
        fn = pl.kernel(body_w, out_type=tuple(out_type), mesh=mesh,
                       scratch_types=tuple(scratch.values()))
        return fn
    else:
        def body_nw(y, src, dst, et, w, accp, *scr):
            return body(y, src, dst, et, w, accp, None, *scr)
        fn = pl.kernel(body_nw, out_type=tuple(out_type), mesh=mesh,
                       scratch_types=tuple(scratch.values()))
        return fn


BN = 1000


def _tc1_body(x_ref, w_ref, root_ref, b_ref, y_ref, out0_ref):
    xb = x_ref[...]
    out0_ref[...] = (jnp.dot(xb, root_ref[...],
                             preferred_element_type=jnp.float32)
                     + b_ref[...])
    for r in range(8):
        y_ref[r] = jnp.dot(xb, w_ref[r], preferred_element_type=jnp.float32)


def _tc2_body(out0_ref, acc_ref, w_ref, root_ref, b_ref, y_ref, out02_ref):
    h = jnp.maximum(out0_ref[...] + acc_ref[0] + acc_ref[1], 0.0)
    out02_ref[...] = (jnp.dot(h, root_ref[...],
                              preferred_element_type=jnp.float32)
                      + b_ref[...])
    for r in range(8):
        y_ref[r] = jnp.dot(h, w_ref[r], preferred_element_type=jnp.float32)


def _tc3_body(out0_ref, acc_ref, out_ref):
    out_ref[...] = out0_ref[...] + acc_ref[0] + acc_ref[1]


def _tc_layer1(x, W, root, b, N):
    grid = (N // BN,)
    return pl.pallas_call(
        _tc1_body,
        grid=grid,
        in_specs=[
            pl.BlockSpec((BN, 128), lambda i: (i, 0)),
            pl.BlockSpec((8, 128, 128), lambda i: (0, 0, 0)),
            pl.BlockSpec((128, 128), lambda i: (0, 0)),
            pl.BlockSpec((1, 128), lambda i: (0, 0)),
        ],
        out_specs=[
            pl.BlockSpec((8, BN, 128), lambda i: (0, i, 0)),
            pl.BlockSpec((BN, 128), lambda i: (i, 0)),
        ],
        out_shape=[
            jax.ShapeDtypeStruct((8, N, 128), jnp.float32),
            jax.ShapeDtypeStruct((N, 128), jnp.float32),
        ],
    )(x, W, root, b.reshape(1, 128))


def _tc_layer2(out0, accp, W, root, b, N):
    grid = (N // BN,)
    return pl.pallas_call(
        _tc2_body,
        grid=grid,
        in_specs=[
            pl.BlockSpec((BN, 128), lambda i: (i, 0)),
            pl.BlockSpec((2, BN, 128), lambda i: (0, i, 0)),
            pl.BlockSpec((8, 128, 128), lambda i: (0, 0, 0)),
            pl.BlockSpec((128, 128), lambda i: (0, 0)),
            pl.BlockSpec((1, 128), lambda i: (0, 0)),
        ],
        out_specs=[
            pl.BlockSpec((8, BN, 128), lambda i: (0, i, 0)),
            pl.BlockSpec((BN, 128), lambda i: (i, 0)),
        ],
        out_shape=[
            jax.ShapeDtypeStruct((8, N, 128), jnp.float32),
            jax.ShapeDtypeStruct((N, 128), jnp.float32),
        ],
    )(out0, accp, W, root, b.reshape(1, 128))


def _tc_final(out0, accp, N):
    grid = (N // BN,)
    return pl.pallas_call(
        _tc3_body,
        grid=grid,
        in_specs=[
            pl.BlockSpec((BN, 128), lambda i: (i, 0)),
            pl.BlockSpec((2, BN, 128), lambda i: (0, i, 0)),
        ],
        out_specs=pl.BlockSpec((BN, 128), lambda i: (i, 0)),
        out_shape=jax.ShapeDtypeStruct((N, 128), jnp.float32),
    )(out0, accp)


@jax.jit
def kernel(x, edge_index, edge_type, W1, root1, b1, W2, root2, b2):
    N, d = x.shape
    E = edge_index.shape[1]
    R = W1.shape[0]
    src = edge_index[0].astype(jnp.int32)
    dst = edge_index[1].astype(jnp.int32)
    et = edge_type.astype(jnp.int32)

    agg1 = _make_sc_agg(E, N, R, compute_w=True)
    agg2 = _make_sc_agg(E, N, R, compute_w=False)

    y1, out0_1 = _tc_layer1(x, W1, root1, b1, N)
    accp1, w = agg1(y1.reshape(R * N, d), src, dst, et)
    y2, out0_2 = _tc_layer2(out0_1, accp1.reshape(NC, N, d), W2, root2, b2, N)
    accp2, = agg2(y2.reshape(R * N, d), src, dst, et, w)
    return _tc_final(out0_2, accp2.reshape(NC, N, d), N)


# speedup vs baseline: 36.4103x; 2.1065x over previous
import functools

import jax
import jax.numpy as jnp
from jax import lax
from jax.experimental import pallas as pl
from jax.experimental.pallas import tpu as pltpu
from jax.experimental.pallas import tpu_sc as plsc

NC = 2
NS = 16
L = 16

GRP = 80
MAC = 2000
NCH = MAC // GRP


def _zero_f32(ref, n):
    def body(j, _):
        ref[pl.ds(16 * j, 16)] = jnp.zeros((16,), jnp.float32)
        return 0
    lax.fori_loop(0, n // 16, body, 0)


def _make_sc_agg(E, N, R, compute_w):
    EPT = E // (NC * NS)
    assert EPT % MAC == 0
    NMAC = EPT // MAC
    EPC = E // NS
    assert EPC % MAC == 0
    NMAC_CNT = EPC // MAC
    ZB = 1280
    CNT_PAD = ((R * N + NS * ZB - 1) // (NS * ZB)) * (NS * ZB)
    ZROWS = CNT_PAD // NS
    assert ZROWS % ZB == 0
    ACK = (N // NS) & ~7
    REM = N - ACK * NS

    mesh = plsc.VectorSubcoreMesh(core_axis_name="c", subcore_axis_name="s",
                                  num_cores=NC, num_subcores=NS)

    out_type = [jax.ShapeDtypeStruct((NC * N, 128), jnp.float32)]
    if compute_w:
        out_type.append(jax.ShapeDtypeStruct((E,), jnp.float32))

    scratch = dict(
        acc_sh=pltpu.VMEM_SHARED((N, 128), jnp.float32),
        rows_v=pltpu.VMEM((2, GRP, 128), jnp.float32),
        etv=pltpu.VMEM((MAC,), jnp.int32),
        srcv=pltpu.VMEM((MAC,), jnp.int32),
        dstv=pltpu.VMEM((NCH, GRP), jnp.int32),
        gidx=pltpu.VMEM((NCH, GRP), jnp.int32),
        wv=pltpu.VMEM((MAC,), jnp.float32),
        zb=pltpu.VMEM((ZB,), jnp.float32),
        sem_g=pltpu.SemaphoreType.DMA((2,)),
        sem_s=pltpu.SemaphoreType.DMA((2,)),
        sem_c=pltpu.SemaphoreType.DMA,
    )
    if compute_w:
        scratch.update(
            cnt_sh=pltpu.VMEM_SHARED((CNT_PAD,), jnp.float32),
            ones_v=pltpu.VMEM((GRP,), jnp.float32),
        )

    def body(y_hbm, src_hbm, dst_hbm, et_hbm, w_hbm, accp_hbm, w_out_hbm,
             acc_sh, rows_v, etv, srcv, dstv, gidx, wv, zb,
             sem_g, sem_s, sem_c,
             cnt_sh=None, ones_v=None):
        c = lax.axis_index("c")
        s = lax.axis_index("s")
        tile = c * NS + s
        ebase = tile * EPT

        def zrow(j, _):
            for q in range(8):
                rows_v[0, j, pl.ds(16 * q, 16)] = jnp.zeros((16,), jnp.float32)
            return 0
        lax.fori_loop(0, GRP, zrow, 0)
        a0 = s * ACK
        done = 0
        while done < ACK:
            n = min(GRP, ACK - done)
            pltpu.sync_copy(rows_v.at[0, pl.ds(0, n)],
                            acc_sh.at[pl.ds(a0 + done, n)])
            done += n
        if REM:
            @pl.when(s == 0)
            def _():
                pltpu.sync_copy(rows_v.at[0, pl.ds(0, REM)],
                                acc_sh.at[pl.ds(NS * ACK, REM)])
        if compute_w:
            _zero_f32(zb, ZB)
            for zi in range(ZROWS // ZB):
                pltpu.sync_copy(zb, cnt_sh.at[pl.ds(s * ZROWS + zi * ZB, ZB)])
            def ofill(j, _):
                ones_v[pl.ds(16 * j, 16)] = jnp.ones((16,), jnp.float32)
                return 0
            lax.fori_loop(0, GRP // 16, ofill, 0)
        plsc.subcore_barrier()

        def load_dst(base):
            def issue(ch, _):
                pltpu.async_copy(dst_hbm.at[pl.ds(base + GRP * ch, GRP)],
                                 dstv.at[ch], sem_c)
                return 0
            lax.fori_loop(0, NCH, issue, 0)
            def drain(ch, _):
                pltpu.make_async_copy(dst_hbm.at[pl.ds(base + GRP * ch, GRP)],
                                      dstv.at[ch], sem_c).wait()
                return 0
            lax.fori_loop(0, NCH, drain, 0)

        def load_cidx(base):
            pltpu.sync_copy(et_hbm.at[pl.ds(base, MAC)], etv)
            load_dst(base)
            def cg2(ch, _):
                def cg(jj, _):
                    col = 16 * jj
                    e16 = etv[pl.ds(GRP * ch + col, 16)]
                    d16 = dstv[ch, pl.ds(col, 16)]
                    gidx[ch, pl.ds(col, 16)] = e16 * N + d16
                    return 0
                lax.fori_loop(0, GRP // 16, cg, 0)
                return 0
            lax.fori_loop(0, NCH, cg2, 0)

        if compute_w:
            def count_mac(m, _):
                base = s * EPC + m * MAC
                load_cidx(base)
                def issue(ch, _):
                    pltpu.async_copy(ones_v, cnt_sh.at[gidx.at[ch]], sem_c,
                                     add=True)
                    return 0
                lax.fori_loop(0, NCH, issue, 0)
                def drain(ch, _):
                    pltpu.make_async_copy(ones_v, cnt_sh.at[gidx.at[ch]],
                                          sem_c).wait()
                    return 0
                lax.fori_loop(0, NCH, drain, 0)
                return 0
            lax.fori_loop(0, NMAC_CNT, count_mac, 0)
            plsc.subcore_barrier()

            def w_mac(m, _):
                base = ebase + m * MAC
                load_cidx(base)
                def issue(ch, _):
                    pltpu.async_copy(cnt_sh.at[gidx.at[ch]],
                                     wv.at[pl.ds(GRP * ch, GRP)], sem_c)
                    return 0
                lax.fori_loop(0, NCH, issue, 0)
                def drain(ch, _):
                    pltpu.make_async_copy(cnt_sh.at[gidx.at[ch]],
                                          wv.at[pl.ds(GRP * ch, GRP)],
                                          sem_c).wait()
                    return 0
                lax.fori_loop(0, NCH, drain, 0)
                def wcomp(j, _):
                    cvec = wv[pl.ds(16 * j, 16)]
                    wv[pl.ds(16 * j, 16)] = 1.0 / jnp.maximum(cvec, 1.0)
                    return 0
                lax.fori_loop(0, MAC // 16, wcomp, 0)
                pltpu.sync_copy(wv, w_out_hbm.at[pl.ds(base, MAC)])
                return 0
            lax.fori_loop(0, NMAC, w_mac, 0)
            w_src = w_out_hbm
        else:
            w_src = w_hbm

        def idx_pass(ch):
            def ig(jj, _):
                col = 16 * jj
                e16 = etv[pl.ds(GRP * ch + col, 16)]
                s16 = srcv[pl.ds(GRP * ch + col, 16)]
                gidx[ch, pl.ds(col, 16)] = e16 * N + s16
                return 0
            lax.fori_loop(0, GRP // 16, ig, 0)

        def scale_pass(p, ch):
            def scale(jj, _):
                w16 = wv[pl.ds(GRP * ch + 16 * jj, 16)]
                for t in range(16):
                    wb = w16.at[jnp.full((16,), t, jnp.int32)].get(
                        mode="promise_in_bounds")
                    j = 16 * jj + t
                    for q in range(8):
                        rows_v[p, j, pl.ds(16 * q, 16)] = (
                            rows_v[p, j, pl.ds(16 * q, 16)] * wb)
                return 0
            lax.fori_loop(0, GRP // 16, scale, 0)

        def issue_gather(ch, p):
            pltpu.async_copy(y_hbm.at[gidx.at[ch]], rows_v.at[p],
                             sem_g.at[p])

        def wait_gather(ch, p):
            pltpu.make_async_copy(y_hbm.at[gidx.at[ch]], rows_v.at[p],
                                  sem_g.at[p]).wait()

        def issue_scat(ch, p):
            pltpu.async_copy(rows_v.at[p], acc_sh.at[dstv.at[ch]],
                             sem_s.at[p], add=True)

        def wait_scat(ch, p):
            pltpu.make_async_copy(rows_v.at[p], acc_sh.at[dstv.at[ch]],
                                  sem_s.at[p]).wait()

        assert NCH % 2 == 1
        def main_mac(m, _):
            base = ebase + m * MAC
            pltpu.sync_copy(et_hbm.at[pl.ds(base, MAC)], etv)
            pltpu.sync_copy(src_hbm.at[pl.ds(base, MAC)], srcv)
            pltpu.sync_copy(w_src.at[pl.ds(base, MAC)], wv)
            load_dst(base)
            idx_pass(0)
            issue_gather(0, 0)
            def piter(i, _):
                c0 = 2 * i
                idx_pass(c0 + 1)
                @pl.when(i > 0)
                def _():
                    wait_scat(c0 - 1, 1)
                issue_gather(c0 + 1, 1)
                wait_gather(c0, 0)
                scale_pass(0, c0)
                issue_scat(c0, 0)
                idx_pass(c0 + 2)
                wait_scat(c0, 0)
                issue_gather(c0 + 2, 0)
                wait_gather(c0 + 1, 1)
                scale_pass(1, c0 + 1)
                issue_scat(c0 + 1, 1)
                return 0
            lax.fori_loop(0, (NCH - 1) // 2, piter, 0)
            last = NCH - 1
            wait_scat(last - 1, 1)
            wait_gather(last, 0)
            scale_pass(0, last)
            issue_scat(last, 0)
            wait_scat(last, 0)
            return 0
        lax.fori_loop(0, NMAC, main_mac, 0)
        plsc.subcore_barrier()

        done = 0
        while done < ACK:
            n = min(GRP, ACK - done)
            pltpu.sync_copy(acc_sh.at[pl.ds(a0 + done, n)],
                            accp_hbm.at[pl.ds(c * N + a0 + done, n)])
            done += n
        if REM:
            @pl.when(s == 0)
            def _():
                pltpu.sync_copy(acc_sh.at[pl.ds(NS * ACK, REM)],
                                accp_hbm.at[pl.ds(c * N + NS * ACK, REM)])

    if compute_w:
        def body_w(y, src, dst, et, accp, w_out, *scr):
            return body(y, src, dst, et, None, accp, w_out, *scr)
        fn = pl.kernel(body_w, out_type=tuple(out_type), mesh=mesh,
                       scratch_types=tuple(scratch.values()))
        return fn
    else:
        def body_nw(y, src, dst, et, w, accp, *scr):
            return body(y, src, dst, et, w, accp, None, *scr)
        fn = pl.kernel(body_nw, out_type=tuple(out_type), mesh=mesh,
                       scratch_types=tuple(scratch.values()))
        return fn


BN = 1000


def _tc1_body(x_ref, w_ref, root_ref, b_ref, y_ref, out0_ref):
    xb = x_ref[...]
    out0_ref[...] = (jnp.dot(xb, root_ref[...],
                             preferred_element_type=jnp.float32)
                     + b_ref[...])
    for r in range(8):
        y_ref[r] = jnp.dot(xb, w_ref[r], preferred_element_type=jnp.float32)


def _tc2_body(out0_ref, acc_ref, w_ref, root_ref, b_ref, y_ref, out02_ref):
    h = jnp.maximum(out0_ref[...] + acc_ref[0] + acc_ref[1], 0.0)
    out02_ref[...] = (jnp.dot(h, root_ref[...],
                              preferred_element_type=jnp.float32)
                      + b_ref[...])
    for r in range(8):
        y_ref[r] = jnp.dot(h, w_ref[r], preferred_element_type=jnp.float32)


def _tc3_body(out0_ref, acc_ref, out_ref):
    out_ref[...] = out0_ref[...] + acc_ref[0] + acc_ref[1]


def _tc_layer1(x, W, root, b, N):
    grid = (N // BN,)
    return pl.pallas_call(
        _tc1_body,
        grid=grid,
        in_specs=[
            pl.BlockSpec((BN, 128), lambda i: (i, 0)),
            pl.BlockSpec((8, 128, 128), lambda i: (0, 0, 0)),
            pl.BlockSpec((128, 128), lambda i: (0, 0)),
            pl.BlockSpec((1, 128), lambda i: (0, 0)),
        ],
        out_specs=[
            pl.BlockSpec((8, BN, 128), lambda i: (0, i, 0)),
            pl.BlockSpec((BN, 128), lambda i: (i, 0)),
        ],
        out_shape=[
            jax.ShapeDtypeStruct((8, N, 128), jnp.float32),
            jax.ShapeDtypeStruct((N, 128), jnp.float32),
        ],
    )(x, W, root, b.reshape(1, 128))


def _tc_layer2(out0, accp, W, root, b, N):
    grid = (N // BN,)
    return pl.pallas_call(
        _tc2_body,
        grid=grid,
        in_specs=[
            pl.BlockSpec((BN, 128), lambda i: (i, 0)),
            pl.BlockSpec((2, BN, 128), lambda i: (0, i, 0)),
            pl.BlockSpec((8, 128, 128), lambda i: (0, 0, 0)),
            pl.BlockSpec((128, 128), lambda i: (0, 0)),
            pl.BlockSpec((1, 128), lambda i: (0, 0)),
        ],
        out_specs=[
            pl.BlockSpec((8, BN, 128), lambda i: (0, i, 0)),
            pl.BlockSpec((BN, 128), lambda i: (i, 0)),
        ],
        out_shape=[
            jax.ShapeDtypeStruct((8, N, 128), jnp.float32),
            jax.ShapeDtypeStruct((N, 128), jnp.float32),
        ],
    )(out0, accp, W, root, b.reshape(1, 128))


def _tc_final(out0, accp, N):
    grid = (N // BN,)
    return pl.pallas_call(
        _tc3_body,
        grid=grid,
        in_specs=[
            pl.BlockSpec((BN, 128), lambda i: (i, 0)),
            pl.BlockSpec((2, BN, 128), lambda i: (0, i, 0)),
        ],
        out_specs=pl.BlockSpec((BN, 128), lambda i: (i, 0)),
        out_shape=jax.ShapeDtypeStruct((N, 128), jnp.float32),
    )(out0, accp)


@jax.jit
def kernel(x, edge_index, edge_type, W1, root1, b1, W2, root2, b2):
    N, d = x.shape
    E = edge_index.shape[1]
    R = W1.shape[0]
    src = edge_index[0].astype(jnp.int32)
    dst = edge_index[1].astype(jnp.int32)
    et = edge_type.astype(jnp.int32)

    agg1 = _make_sc_agg(E, N, R, compute_w=True)
    agg2 = _make_sc_agg(E, N, R, compute_w=False)

    y1, out0_1 = _tc_layer1(x, W1, root1, b1, N)
    accp1, w = agg1(y1.reshape(R * N, d), src, dst, et)
    y2, out0_2 = _tc_layer2(out0_1, accp1.reshape(NC, N, d), W2, root2, b2, N)
    accp2, = agg2(y2.reshape(R * N, d), src, dst, et, w)
    return _tc_final(out0_2, accp2.reshape(NC, N, d), N)
